# baseline (device time: 138632 ns/iter reference)
import numpy as np

import jax
import jax.numpy as jnp
from jax import lax
from jax.experimental import pallas as pl
from jax.experimental.pallas import tpu as pltpu

N_DEV = 16
B = 2
SQ = 256
D = 768
HQ = 4
DH = 64
HD = HQ * DH
ROWS = B * SQ


def _rot_mat() -> np.ndarray:
    r = np.zeros((DH, DH), np.float32)
    for i in range(DH // 2):
        r[2 * i + 1, 2 * i] = -1.0
        r[2 * i, 2 * i + 1] = 1.0
    return np.kron(np.eye(HQ, dtype=np.float32), r)


def kernel(x, Wq, Wk, Wv, Wo):
    my_pos = lax.axis_index("i")
    x2d = x.reshape(ROWS, D)

    inv = 1.0 / (10000.0 ** (jnp.arange(0, DH, 2, dtype=jnp.float32) / DH))
    pos = (my_pos * SQ + jnp.arange(SQ, dtype=jnp.float32))[:, None] * inv[None, :]
    cos_big = jnp.tile(jnp.repeat(jnp.cos(pos), 2, axis=1), (B, HQ))
    sin_big = jnp.tile(jnp.repeat(jnp.sin(pos), 2, axis=1), (B, HQ))
    rot = jnp.asarray(_rot_mat(), dtype=jnp.bfloat16)

    def body(x_ref, wq_ref, wk_ref, wv_ref, wo_ref, cos_ref, sin_ref,
             rot_ref, out_ref, kv_ref, send_sems, recv_sems):
        me = lax.axis_index("i")
        left = lax.rem(me + N_DEV - 1, N_DEV)
        right = lax.rem(me + 1, N_DEV)

        xb = x_ref[...].astype(jnp.bfloat16)
        rot_m = rot_ref[...]

        def project(w_ref):
            return jax.lax.dot(xb, w_ref[...].astype(jnp.bfloat16),
                               preferred_element_type=jnp.float32)

        def rope(t):
            tr = jax.lax.dot(t.astype(jnp.bfloat16), rot_m,
                             preferred_element_type=jnp.float32)
            return (t * cos_ref[...] + tr * sin_ref[...]).astype(jnp.bfloat16)

        q = rope(project(wq_ref))
        kv_ref[me, 0, :, :] = rope(project(wk_ref))
        kv_ref[me, 1, :, :] = project(wv_ref).astype(jnp.bfloat16)

        barrier = pltpu.get_barrier_semaphore()
        for nbr in (left, right):
            pl.semaphore_signal(barrier, inc=1, device_id=(nbr,),
                                device_id_type=pl.DeviceIdType.MESH)
        pl.semaphore_wait(barrier, 2)

        sends = []
        for h in range(N_DEV - 1):
            o_s = lax.rem(me - h + 2 * N_DEV, N_DEV)
            o_r = lax.rem(me - h - 1 + 2 * N_DEV, N_DEV)
            send = pltpu.make_async_remote_copy(
                src_ref=kv_ref.at[o_s], dst_ref=kv_ref.at[o_s],
                send_sem=send_sems.at[h], recv_sem=recv_sems.at[h],
                device_id=(right,), device_id_type=pl.DeviceIdType.MESH)
            send.start()
            sends.append(send)
            recv = pltpu.make_async_remote_copy(
                src_ref=kv_ref.at[o_s], dst_ref=kv_ref.at[o_r],
                send_sem=send_sems.at[h], recv_sem=recv_sems.at[h],
                device_id=(left,), device_id_type=pl.DeviceIdType.MESH)
            recv.wait_recv()
        for s in sends:
            s.wait_send()

        ctx_rows = []
        for b in range(B):
            ctx_h = []
            for hh in range(HQ):
                rs, cs = b * SQ, hh * DH
                qbh = q[rs:rs + SQ, cs:cs + DH]
                kbh = jnp.concatenate(
                    [kv_ref[o, 0, rs:rs + SQ, cs:cs + DH] for o in range(N_DEV)],
                    axis=0)
                vbh = jnp.concatenate(
                    [kv_ref[o, 1, rs:rs + SQ, cs:cs + DH] for o in range(N_DEV)],
                    axis=0)
                s = jax.lax.dot_general(
                    qbh, kbh, (((1,), (1,)), ((), ())),
                    preferred_element_type=jnp.float32) * 0.125
                m = jnp.max(s, axis=1, keepdims=True)
                w = jnp.exp(s - m)
                w = w / jnp.sum(w, axis=1, keepdims=True)
                ctx_h.append(jax.lax.dot(w.astype(jnp.bfloat16), vbh,
                                         preferred_element_type=jnp.float32))
            ctx_rows.append(jnp.concatenate(ctx_h, axis=1))
        ctx = jnp.concatenate(ctx_rows, axis=0).astype(jnp.bfloat16)
        out_ref[...] = jax.lax.dot(ctx, wo_ref[...].astype(jnp.bfloat16),
                                   preferred_element_type=jnp.float32)

    out2d = pl.pallas_call(
        body,
        out_shape=jax.ShapeDtypeStruct((ROWS, D), jnp.float32),
        in_specs=[pl.BlockSpec(memory_space=pltpu.VMEM)] * 8,
        out_specs=pl.BlockSpec(memory_space=pltpu.VMEM),
        scratch_shapes=[
            pltpu.VMEM((N_DEV, 2, ROWS, HD), jnp.bfloat16),
            pltpu.SemaphoreType.DMA((N_DEV - 1,)),
            pltpu.SemaphoreType.DMA((N_DEV - 1,)),
        ],
        compiler_params=pltpu.CompilerParams(collective_id=0),
    )(x2d, Wq, Wk, Wv, Wo, cos_big, sin_big, rot)
    return out2d.reshape(B, SQ, D)


# device time: 98737 ns/iter; 1.4041x vs baseline; 1.4041x over previous
import numpy as np

import jax
import jax.numpy as jnp
from jax import lax
from jax.experimental import pallas as pl
from jax.experimental.pallas import tpu as pltpu

N_DEV = 16
B = 2
SQ = 256
D = 768
HQ = 4
DH = 64
HD = HQ * DH
ROWS = B * SQ


def _rot_mat() -> np.ndarray:
    r = np.zeros((DH, DH), np.float32)
    for i in range(DH // 2):
        r[2 * i + 1, 2 * i] = -1.0
        r[2 * i, 2 * i + 1] = 1.0
    return np.kron(np.eye(HQ, dtype=np.float32), r)


def kernel(x, Wq, Wk, Wv, Wo):
    my_pos = lax.axis_index("i")
    x2d = x.reshape(ROWS, D)

    inv = 1.0 / (10000.0 ** (jnp.arange(0, DH, 2, dtype=jnp.float32) / DH))
    pos = (my_pos * SQ + jnp.arange(SQ, dtype=jnp.float32))[:, None] * inv[None, :]
    cos_big = jnp.tile(jnp.repeat(jnp.cos(pos), 2, axis=1), (B, HQ))
    sin_big = jnp.tile(jnp.repeat(jnp.sin(pos), 2, axis=1), (B, HQ))
    rot = jnp.asarray(_rot_mat(), dtype=jnp.bfloat16)

    def body(x_ref, wq_ref, wk_ref, wv_ref, wo_ref, cos_ref, sin_ref,
             rot_ref, out_ref, kv_ref, send_r, recv_r, send_l, recv_l):
        me = lax.axis_index("i")
        left = lax.rem(me + N_DEV - 1, N_DEV)
        right = lax.rem(me + 1, N_DEV)

        xb = x_ref[...].astype(jnp.bfloat16)
        rot_m = rot_ref[...]

        def project(w_ref):
            return jax.lax.dot(xb, w_ref[...].astype(jnp.bfloat16),
                               preferred_element_type=jnp.float32)

        def rope(t):
            tr = jax.lax.dot(t.astype(jnp.bfloat16), rot_m,
                             preferred_element_type=jnp.float32)
            return (t * cos_ref[...] + tr * sin_ref[...]).astype(jnp.bfloat16)

        q = rope(project(wq_ref))
        kv_ref[me, 0, :, :] = rope(project(wk_ref))
        kv_ref[me, 1, :, :] = project(wv_ref).astype(jnp.bfloat16)

        barrier = pltpu.get_barrier_semaphore()
        for nbr in (left, right):
            pl.semaphore_signal(barrier, inc=1, device_id=(nbr,),
                                device_id_type=pl.DeviceIdType.MESH)
        pl.semaphore_wait(barrier, 2)

        HOPS_R, HOPS_L = 8, 7

        def hop(h, sem_s, sem_r, dst_dev, src_dev, off):
            o_s = lax.rem(me + off * h + 2 * N_DEV, N_DEV)
            o_r = lax.rem(me + off * (h + 1) + 2 * N_DEV, N_DEV)
            send = pltpu.make_async_remote_copy(
                src_ref=kv_ref.at[o_s], dst_ref=kv_ref.at[o_s],
                send_sem=sem_s.at[h], recv_sem=sem_r.at[h],
                device_id=(dst_dev,), device_id_type=pl.DeviceIdType.MESH)
            send.start()
            recv = pltpu.make_async_remote_copy(
                src_ref=kv_ref.at[o_s], dst_ref=kv_ref.at[o_r],
                send_sem=sem_s.at[h], recv_sem=sem_r.at[h],
                device_id=(src_dev,), device_id_type=pl.DeviceIdType.MESH)
            return send, recv

        sends = []
        for h in range(HOPS_R):
            s_r, r_r = hop(h, send_r, recv_r, right, left, -1)
            sends.append(s_r)
            if h < HOPS_L:
                s_l, r_l = hop(h, send_l, recv_l, left, right, +1)
                sends.append(s_l)
            r_r.wait_recv()
            if h < HOPS_L:
                r_l.wait_recv()
        for s in sends:
            s.wait_send()

        ctx_rows = []
        for b in range(B):
            ctx_h = []
            for hh in range(HQ):
                rs, cs = b * SQ, hh * DH
                qbh = q[rs:rs + SQ, cs:cs + DH]
                kbh = jnp.concatenate(
                    [kv_ref[o, 0, rs:rs + SQ, cs:cs + DH] for o in range(N_DEV)],
                    axis=0)
                vbh = jnp.concatenate(
                    [kv_ref[o, 1, rs:rs + SQ, cs:cs + DH] for o in range(N_DEV)],
                    axis=0)
                s = jax.lax.dot_general(
                    qbh, kbh, (((1,), (1,)), ((), ())),
                    preferred_element_type=jnp.float32) * 0.125
                m = jnp.max(s, axis=1, keepdims=True)
                w = jnp.exp(s - m)
                w = w / jnp.sum(w, axis=1, keepdims=True)
                ctx_h.append(jax.lax.dot(w.astype(jnp.bfloat16), vbh,
                                         preferred_element_type=jnp.float32))
            ctx_rows.append(jnp.concatenate(ctx_h, axis=1))
        ctx = jnp.concatenate(ctx_rows, axis=0).astype(jnp.bfloat16)
        out_ref[...] = jax.lax.dot(ctx, wo_ref[...].astype(jnp.bfloat16),
                                   preferred_element_type=jnp.float32)

    out2d = pl.pallas_call(
        body,
        out_shape=jax.ShapeDtypeStruct((ROWS, D), jnp.float32),
        in_specs=[pl.BlockSpec(memory_space=pltpu.VMEM)] * 8,
        out_specs=pl.BlockSpec(memory_space=pltpu.VMEM),
        scratch_shapes=[
            pltpu.VMEM((N_DEV, 2, ROWS, HD), jnp.bfloat16),
            pltpu.SemaphoreType.DMA((8,)),
            pltpu.SemaphoreType.DMA((8,)),
            pltpu.SemaphoreType.DMA((7,)),
            pltpu.SemaphoreType.DMA((7,)),
        ],
        compiler_params=pltpu.CompilerParams(collective_id=0),
    )(x2d, Wq, Wk, Wv, Wo, cos_big, sin_big, rot)
    return out2d.reshape(B, SQ, D)


# device time: 86640 ns/iter; 1.6001x vs baseline; 1.1396x over previous
import numpy as np

import jax
import jax.numpy as jnp
from jax import lax
from jax.experimental import pallas as pl
from jax.experimental.pallas import tpu as pltpu

N_DEV = 16
B = 2
SQ = 256
D = 768
HQ = 4
DH = 64
HD = HQ * DH
ROWS = B * SQ


def _rot_mat() -> np.ndarray:
    r = np.zeros((DH, DH), np.float32)
    for i in range(DH // 2):
        r[2 * i + 1, 2 * i] = -1.0
        r[2 * i, 2 * i + 1] = 1.0
    return np.kron(np.eye(HQ, dtype=np.float32), r)


def kernel(x, Wq, Wk, Wv, Wo):
    my_pos = lax.axis_index("i")
    x2d = x.reshape(ROWS, D)

    inv = 1.0 / (10000.0 ** (jnp.arange(0, DH, 2, dtype=jnp.float32) / DH))
    pos = (my_pos * SQ + jnp.arange(SQ, dtype=jnp.float32))[:, None] * inv[None, :]
    cos_big = jnp.tile(jnp.repeat(jnp.cos(pos), 2, axis=1), (B, HQ))
    sin_big = jnp.tile(jnp.repeat(jnp.sin(pos), 2, axis=1), (B, HQ))
    rot = jnp.asarray(_rot_mat(), dtype=jnp.bfloat16)

    def body(x_ref, wq_ref, wk_ref, wv_ref, wo_ref, cos_ref, sin_ref,
             rot_ref, out_ref, kv_ref, send_r, recv_r, send_l, recv_l):
        me = lax.axis_index("i")
        p = lax.rem(me, 4)
        z = lax.div(me, 4)
        succ = jnp.where(
            p == 0, jnp.where(z < 3, me + 4, me + 1),
            jnp.where(p == 1, jnp.where(z > 0, me - 4, me + 1),
                      jnp.where(p == 2, jnp.where(z < 3, me + 4, me + 1),
                                jnp.where(z > 0, me - 4, me - 3))))
        pred = jnp.where(
            p == 0, jnp.where(z > 0, me - 4, me + 3),
            jnp.where(p == 1, jnp.where(z < 3, me + 4, me - 1),
                      jnp.where(p == 2, jnp.where(z > 0, me - 4, me - 1),
                                jnp.where(z < 3, me + 4, me - 1))))
        c_me = jnp.where(p == 0, z,
                         jnp.where(p == 1, 7 - z,
                                   jnp.where(p == 2, 8 + z, 15 - z)))
        left, right = pred, succ

        xb = x_ref[...].astype(jnp.bfloat16)
        rot_m = rot_ref[...]

        def project(w_ref):
            return jax.lax.dot(xb, w_ref[...].astype(jnp.bfloat16),
                               preferred_element_type=jnp.float32)

        def rope(t):
            tr = jax.lax.dot(t.astype(jnp.bfloat16), rot_m,
                             preferred_element_type=jnp.float32)
            return (t * cos_ref[...] + tr * sin_ref[...]).astype(jnp.bfloat16)

        q = rope(project(wq_ref))
        kv_ref[c_me, 0, :, :] = rope(project(wk_ref))
        kv_ref[c_me, 1, :, :] = project(wv_ref).astype(jnp.bfloat16)

        barrier = pltpu.get_barrier_semaphore()
        for nbr in (left, right):
            pl.semaphore_signal(barrier, inc=1, device_id=(nbr,),
                                device_id_type=pl.DeviceIdType.MESH)
        pl.semaphore_wait(barrier, 2)

        HOPS_R, HOPS_L = 8, 7

        def hop(h, sem_s, sem_r, dst_dev, src_dev, off):
            o_s = lax.rem(c_me + off * h + 2 * N_DEV, N_DEV)
            o_r = lax.rem(c_me + off * (h + 1) + 2 * N_DEV, N_DEV)
            send = pltpu.make_async_remote_copy(
                src_ref=kv_ref.at[o_s], dst_ref=kv_ref.at[o_s],
                send_sem=sem_s.at[h], recv_sem=sem_r.at[h],
                device_id=(dst_dev,), device_id_type=pl.DeviceIdType.MESH)
            send.start()
            recv = pltpu.make_async_remote_copy(
                src_ref=kv_ref.at[o_s], dst_ref=kv_ref.at[o_r],
                send_sem=sem_s.at[h], recv_sem=sem_r.at[h],
                device_id=(src_dev,), device_id_type=pl.DeviceIdType.MESH)
            return send, recv

        sends = []
        for h in range(HOPS_R):
            s_r, r_r = hop(h, send_r, recv_r, right, left, -1)
            sends.append(s_r)
            if h < HOPS_L:
                s_l, r_l = hop(h, send_l, recv_l, left, right, +1)
                sends.append(s_l)
            r_r.wait_recv()
            if h < HOPS_L:
                r_l.wait_recv()
        for s in sends:
            s.wait_send()

        ctx_rows = []
        for b in range(B):
            ctx_h = []
            for hh in range(HQ):
                rs, cs = b * SQ, hh * DH
                qbh = q[rs:rs + SQ, cs:cs + DH]
                kbh = jnp.concatenate(
                    [kv_ref[o, 0, rs:rs + SQ, cs:cs + DH] for o in range(N_DEV)],
                    axis=0)
                vbh = jnp.concatenate(
                    [kv_ref[o, 1, rs:rs + SQ, cs:cs + DH] for o in range(N_DEV)],
                    axis=0)
                s = jax.lax.dot_general(
                    qbh, kbh, (((1,), (1,)), ((), ())),
                    preferred_element_type=jnp.float32) * 0.125
                m = jnp.max(s, axis=1, keepdims=True)
                w = jnp.exp(s - m)
                w = w / jnp.sum(w, axis=1, keepdims=True)
                ctx_h.append(jax.lax.dot(w.astype(jnp.bfloat16), vbh,
                                         preferred_element_type=jnp.float32))
            ctx_rows.append(jnp.concatenate(ctx_h, axis=1))
        ctx = jnp.concatenate(ctx_rows, axis=0).astype(jnp.bfloat16)
        out_ref[...] = jax.lax.dot(ctx, wo_ref[...].astype(jnp.bfloat16),
                                   preferred_element_type=jnp.float32)

    out2d = pl.pallas_call(
        body,
        out_shape=jax.ShapeDtypeStruct((ROWS, D), jnp.float32),
        in_specs=[pl.BlockSpec(memory_space=pltpu.VMEM)] * 8,
        out_specs=pl.BlockSpec(memory_space=pltpu.VMEM),
        scratch_shapes=[
            pltpu.VMEM((N_DEV, 2, ROWS, HD), jnp.bfloat16),
            pltpu.SemaphoreType.DMA((8,)),
            pltpu.SemaphoreType.DMA((8,)),
            pltpu.SemaphoreType.DMA((7,)),
            pltpu.SemaphoreType.DMA((7,)),
        ],
        compiler_params=pltpu.CompilerParams(collective_id=0),
    )(x2d, Wq, Wk, Wv, Wo, cos_big, sin_big, rot)
    return out2d.reshape(B, SQ, D)


# device time: 71486 ns/iter; 1.9393x vs baseline; 1.2120x over previous
import numpy as np

import jax
import jax.numpy as jnp
from jax import lax
from jax.experimental import pallas as pl
from jax.experimental.pallas import tpu as pltpu

N_DEV = 16
B = 2
SQ = 256
D = 768
HQ = 4
DH = 64
HD = HQ * DH
ROWS = B * SQ
HOPS_R, HOPS_L = 8, 7


def _rot_mat() -> np.ndarray:
    r = np.zeros((DH, DH), np.float32)
    for i in range(DH // 2):
        r[2 * i + 1, 2 * i] = -1.0
        r[2 * i, 2 * i + 1] = 1.0
    return np.kron(np.eye(HQ, dtype=np.float32), r)


def kernel(x, Wq, Wk, Wv, Wo):
    my_pos = lax.axis_index("i")
    x2d = x.reshape(ROWS, D)

    inv = 1.0 / (10000.0 ** (jnp.arange(0, DH, 2, dtype=jnp.float32) / DH))
    pos = (my_pos * SQ + jnp.arange(SQ, dtype=jnp.float32))[:, None] * inv[None, :]
    cos_big = jnp.tile(jnp.repeat(jnp.cos(pos), 2, axis=1), (B, HQ))
    sin_big = jnp.tile(jnp.repeat(jnp.sin(pos), 2, axis=1), (B, HQ))
    rot = jnp.asarray(_rot_mat(), dtype=jnp.bfloat16)

    def body(x_ref, wq_ref, wk_ref, wv_ref, wo_ref, cos_ref, sin_ref,
             rot_ref, out_ref, kv_ref, send_r, recv_r, send_l, recv_l):
        me = lax.axis_index("i")
        p = lax.rem(me, 4)
        z = lax.div(me, 4)
        succ = jnp.where(
            p == 0, jnp.where(z < 3, me + 4, me + 1),
            jnp.where(p == 1, jnp.where(z > 0, me - 4, me + 1),
                      jnp.where(p == 2, jnp.where(z < 3, me + 4, me + 1),
                                jnp.where(z > 0, me - 4, me - 3))))
        pred = jnp.where(
            p == 0, jnp.where(z > 0, me - 4, me + 3),
            jnp.where(p == 1, jnp.where(z < 3, me + 4, me - 1),
                      jnp.where(p == 2, jnp.where(z > 0, me - 4, me - 1),
                                jnp.where(z < 3, me + 4, me - 1))))
        c_me = jnp.where(p == 0, z,
                         jnp.where(p == 1, 7 - z,
                                   jnp.where(p == 2, 8 + z, 15 - z)))

        xb = x_ref[...].astype(jnp.bfloat16)
        rot_m = rot_ref[...]

        def project(w_ref):
            return jax.lax.dot(xb, w_ref[...].astype(jnp.bfloat16),
                               preferred_element_type=jnp.float32)

        def rope(t):
            tr = jax.lax.dot(t.astype(jnp.bfloat16), rot_m,
                             preferred_element_type=jnp.float32)
            return (t * cos_ref[...] + tr * sin_ref[...]).astype(jnp.bfloat16)

        q = rope(project(wq_ref))
        kv_ref[c_me, 0, :, :] = rope(project(wk_ref))
        kv_ref[c_me, 1, :, :] = project(wv_ref).astype(jnp.bfloat16)

        barrier = pltpu.get_barrier_semaphore()
        for nbr in (pred, succ):
            pl.semaphore_signal(barrier, inc=1, device_id=(nbr,),
                                device_id_type=pl.DeviceIdType.MESH)
        pl.semaphore_wait(barrier, 2)

        def mk(o_src, o_dst, h, part, sems_s, sems_r, dev):
            return pltpu.make_async_remote_copy(
                src_ref=kv_ref.at[o_src, part], dst_ref=kv_ref.at[o_dst, part],
                send_sem=sems_s.at[h, part], recv_sem=sems_r.at[h, part],
                device_id=(dev,), device_id_type=pl.DeviceIdType.MESH)

        def cyc(k):
            return lax.rem(c_me + k + 2 * N_DEV, N_DEV)

        sends = []

        def start_hop(h):
            if h < HOPS_R:
                for part in (0, 1):
                    s = mk(cyc(-h), cyc(-h), h, part, send_r, recv_r, succ)
                    s.start()
                    sends.append(s)
            if h < HOPS_L:
                for part in (0, 1):
                    s = mk(cyc(h), cyc(h), h, part, send_l, recv_l, pred)
                    s.start()
                    sends.append(s)

        qbh = [[q[b * SQ:(b + 1) * SQ, hh * DH:(hh + 1) * DH]
                for hh in range(HQ)] for b in range(B)]
        ms = [[None] * HQ for _ in range(B)]
        ls = [[None] * HQ for _ in range(B)]
        accs = [[None] * HQ for _ in range(B)]

        def attn_update(origins):
            for b in range(B):
                rs = b * SQ
                for hh in range(HQ):
                    cs = hh * DH
                    k = jnp.concatenate(
                        [kv_ref[o, 0, rs:rs + SQ, cs:cs + DH] for o in origins],
                        axis=0)
                    v = jnp.concatenate(
                        [kv_ref[o, 1, rs:rs + SQ, cs:cs + DH] for o in origins],
                        axis=0)
                    s = jax.lax.dot_general(
                        qbh[b][hh], k, (((1,), (1,)), ((), ())),
                        preferred_element_type=jnp.float32) * 0.125
                    m_c = jnp.max(s, axis=1, keepdims=True)
                    if ms[b][hh] is None:
                        m_new = m_c
                        w = jnp.exp(s - m_new)
                        ls[b][hh] = jnp.sum(w, axis=1, keepdims=True)
                        accs[b][hh] = jax.lax.dot(
                            w.astype(jnp.bfloat16), v,
                            preferred_element_type=jnp.float32)
                    else:
                        m_new = jnp.maximum(ms[b][hh], m_c)
                        alpha = jnp.exp(ms[b][hh] - m_new)
                        w = jnp.exp(s - m_new)
                        ls[b][hh] = ls[b][hh] * alpha + jnp.sum(
                            w, axis=1, keepdims=True)
                        accs[b][hh] = accs[b][hh] * alpha + jax.lax.dot(
                            w.astype(jnp.bfloat16), v,
                            preferred_element_type=jnp.float32)
                    ms[b][hh] = m_new

        start_hop(0)
        attn_update([c_me])
        for h in range(HOPS_R):
            for part in (0, 1):
                mk(cyc(-h - 1), cyc(-h - 1), h, part,
                   send_r, recv_r, pred).wait_recv()
            if h < HOPS_L:
                for part in (0, 1):
                    mk(cyc(h + 1), cyc(h + 1), h, part,
                       send_l, recv_l, succ).wait_recv()
            start_hop(h + 1)
            origins = [cyc(-h - 1)] + ([cyc(h + 1)] if h < HOPS_L else [])
            attn_update(origins)
        for s in sends:
            s.wait_send()

        ctx = jnp.concatenate(
            [jnp.concatenate([accs[b][hh] / ls[b][hh] for hh in range(HQ)],
                             axis=1) for b in range(B)],
            axis=0).astype(jnp.bfloat16)
        out_ref[...] = jax.lax.dot(ctx, wo_ref[...].astype(jnp.bfloat16),
                                   preferred_element_type=jnp.float32)

    out2d = pl.pallas_call(
        body,
        out_shape=jax.ShapeDtypeStruct((ROWS, D), jnp.float32),
        in_specs=[pl.BlockSpec(memory_space=pltpu.VMEM)] * 8,
        out_specs=pl.BlockSpec(memory_space=pltpu.VMEM),
        scratch_shapes=[
            pltpu.VMEM((N_DEV, 2, ROWS, HD), jnp.bfloat16),
            pltpu.SemaphoreType.DMA((HOPS_R, 2)),
            pltpu.SemaphoreType.DMA((HOPS_R, 2)),
            pltpu.SemaphoreType.DMA((HOPS_L, 2)),
            pltpu.SemaphoreType.DMA((HOPS_L, 2)),
        ],
        compiler_params=pltpu.CompilerParams(collective_id=0),
    )(x2d, Wq, Wk, Wv, Wo, cos_big, sin_big, rot)
    return out2d.reshape(B, SQ, D)


# device time: 58860 ns/iter; 2.3553x vs baseline; 1.2145x over previous
import numpy as np

import jax
import jax.numpy as jnp
from jax import lax
from jax.experimental import pallas as pl
from jax.experimental.pallas import tpu as pltpu

N_DEV = 16
B = 2
SQ = 256
D = 768
HQ = 4
DH = 64
HD = HQ * DH
ROWS = B * SQ
HOPS_R, HOPS_L = 8, 7


def _rot_mat() -> np.ndarray:
    r = np.zeros((DH, DH), np.float32)
    for i in range(DH // 2):
        r[2 * i + 1, 2 * i] = -1.0
        r[2 * i, 2 * i + 1] = 1.0
    return np.kron(np.eye(HQ, dtype=np.float32), r)


def kernel(x, Wq, Wk, Wv, Wo):
    my_pos = lax.axis_index("i")
    x2d = x.reshape(ROWS, D)

    inv = 1.0 / (10000.0 ** (jnp.arange(0, DH, 2, dtype=jnp.float32) / DH))
    pos = (my_pos * SQ + jnp.arange(SQ, dtype=jnp.float32))[:, None] * inv[None, :]
    cos_big = jnp.tile(jnp.repeat(jnp.cos(pos), 2, axis=1), (B, HQ))
    sin_big = jnp.tile(jnp.repeat(jnp.sin(pos), 2, axis=1), (B, HQ))
    rot = jnp.asarray(_rot_mat(), dtype=jnp.bfloat16)

    def body(x_ref, wq_ref, wk_ref, wv_ref, wo_ref, cos_ref, sin_ref,
             rot_ref, out_ref, kv_ref, send_r, recv_r, send_l, recv_l):
        me = lax.axis_index("i")
        p = lax.rem(me, 4)
        z = lax.div(me, 4)
        succ = jnp.where(
            p == 0, jnp.where(z < 3, me + 4, me + 1),
            jnp.where(p == 1, jnp.where(z > 0, me - 4, me + 1),
                      jnp.where(p == 2, jnp.where(z < 3, me + 4, me + 1),
                                jnp.where(z > 0, me - 4, me - 3))))
        pred = jnp.where(
            p == 0, jnp.where(z > 0, me - 4, me + 3),
            jnp.where(p == 1, jnp.where(z < 3, me + 4, me - 1),
                      jnp.where(p == 2, jnp.where(z > 0, me - 4, me - 1),
                                jnp.where(z < 3, me + 4, me - 1))))
        c_me = jnp.where(p == 0, z,
                         jnp.where(p == 1, 7 - z,
                                   jnp.where(p == 2, 8 + z, 15 - z)))

        xb = x_ref[...].astype(jnp.bfloat16)
        rot_m = rot_ref[...]

        def project(w_ref):
            return jax.lax.dot(xb, w_ref[...].astype(jnp.bfloat16),
                               preferred_element_type=jnp.float32)

        def rope(t):
            tr = jax.lax.dot(t.astype(jnp.bfloat16), rot_m,
                             preferred_element_type=jnp.float32)
            return (t * cos_ref[...] + tr * sin_ref[...]).astype(jnp.bfloat16)

        q = rope(project(wq_ref))
        k_own = rope(project(wk_ref))
        v_own = project(wv_ref).astype(jnp.bfloat16)
        kv_ref[c_me, 0, :, :] = k_own[:SQ]
        kv_ref[c_me, 1, :, :] = k_own[SQ:]
        kv_ref[c_me, 2, :, :] = v_own[:SQ]
        kv_ref[c_me, 3, :, :] = v_own[SQ:]

        barrier = pltpu.get_barrier_semaphore()
        for nbr in (pred, succ):
            pl.semaphore_signal(barrier, inc=1, device_id=(nbr,),
                                device_id_type=pl.DeviceIdType.MESH)
        pl.semaphore_wait(barrier, 2)

        def mk(o, h, part, sems_s, sems_r, dev):
            return pltpu.make_async_remote_copy(
                src_ref=kv_ref.at[o, part], dst_ref=kv_ref.at[o, part],
                send_sem=sems_s.at[h, part], recv_sem=sems_r.at[h, part],
                device_id=(dev,), device_id_type=pl.DeviceIdType.MESH)

        def cyc(k):
            return lax.rem(c_me + k + 2 * N_DEV, N_DEV)

        sends = []

        def start(o, h, part, sems_s, sems_r, dev):
            s = mk(o, h, part, sems_s, sems_r, dev)
            s.start()
            sends.append(s)

        qbh = [[q[b * SQ:(b + 1) * SQ, hh * DH:(hh + 1) * DH]
                for hh in range(HQ)] for b in range(B)]
        ms = [[None] * HQ for _ in range(B)]
        ls = [[None] * HQ for _ in range(B)]
        accs = [[None] * HQ for _ in range(B)]

        def attn_update(origins):
            for b in range(B):
                k_all = jnp.concatenate([kv_ref[o, b] for o in origins], axis=0)
                v_all = jnp.concatenate([kv_ref[o, 2 + b] for o in origins],
                                        axis=0)
                for hh in range(HQ):
                    cs = hh * DH
                    k = k_all[:, cs:cs + DH]
                    v = v_all[:, cs:cs + DH]
                    s = jax.lax.dot_general(
                        qbh[b][hh], k, (((1,), (1,)), ((), ())),
                        preferred_element_type=jnp.float32) * 0.125
                    m_c = jnp.max(s, axis=1, keepdims=True)
                    if ms[b][hh] is None:
                        m_new = m_c
                        w = jnp.exp(s - m_new)
                        ls[b][hh] = jnp.sum(w, axis=1, keepdims=True)
                        accs[b][hh] = jax.lax.dot(
                            w.astype(jnp.bfloat16), v,
                            preferred_element_type=jnp.float32)
                    else:
                        m_new = jnp.maximum(ms[b][hh], m_c)
                        alpha = jnp.exp(ms[b][hh] - m_new)
                        w = jnp.exp(s - m_new)
                        ls[b][hh] = ls[b][hh] * alpha + jnp.sum(
                            w, axis=1, keepdims=True)
                        accs[b][hh] = accs[b][hh] * alpha + jax.lax.dot(
                            w.astype(jnp.bfloat16), v,
                            preferred_element_type=jnp.float32)
                    ms[b][hh] = m_new

        for part in range(4):
            start(c_me, 0, part, send_r, recv_r, succ)
            start(c_me, 0, part, send_l, recv_l, pred)
        attn_update([c_me])
        for h in range(HOPS_R):
            o_r, o_l = cyc(-h - 1), cyc(h + 1)
            for part in range(4):
                mk(o_r, h, part, send_r, recv_r, pred).wait_recv()
                if h + 1 < HOPS_R:
                    start(o_r, h + 1, part, send_r, recv_r, succ)
                if h < HOPS_L:
                    mk(o_l, h, part, send_l, recv_l, succ).wait_recv()
                    if h + 1 < HOPS_L:
                        start(o_l, h + 1, part, send_l, recv_l, pred)
            attn_update([o_r] + ([o_l] if h < HOPS_L else []))
        for s in sends:
            s.wait_send()

        ctx = jnp.concatenate(
            [jnp.concatenate([accs[b][hh] / ls[b][hh] for hh in range(HQ)],
                             axis=1) for b in range(B)],
            axis=0).astype(jnp.bfloat16)
        out_ref[...] = jax.lax.dot(ctx, wo_ref[...].astype(jnp.bfloat16),
                                   preferred_element_type=jnp.float32)

    out2d = pl.pallas_call(
        body,
        out_shape=jax.ShapeDtypeStruct((ROWS, D), jnp.float32),
        in_specs=[pl.BlockSpec(memory_space=pltpu.VMEM)] * 8,
        out_specs=pl.BlockSpec(memory_space=pltpu.VMEM),
        scratch_shapes=[
            pltpu.VMEM((N_DEV, 4, SQ, HD), jnp.bfloat16),
            pltpu.SemaphoreType.DMA((HOPS_R, 4)),
            pltpu.SemaphoreType.DMA((HOPS_R, 4)),
            pltpu.SemaphoreType.DMA((HOPS_L, 4)),
            pltpu.SemaphoreType.DMA((HOPS_L, 4)),
        ],
        compiler_params=pltpu.CompilerParams(collective_id=0),
    )(x2d, Wq, Wk, Wv, Wo, cos_big, sin_big, rot)
    return out2d.reshape(B, SQ, D)
